# Initial kernel scaffold; baseline (speedup 1.0000x reference)
import jax
import jax.numpy as jnp
from jax.experimental import pallas as pl


def kernel(x, w1, b1, w2, b2):
    raise NotImplementedError("write your pallas kernel here")


# speedup vs baseline: 1.0730x; 1.0730x over previous
import functools

import jax
import jax.numpy as jnp
from jax.experimental import pallas as pl
from jax.experimental.pallas import tpu as pltpu


def _attn_kernel(x_ref, w1_ref, b1_ref, w2_ref, ctx_ref, attw_ref, *, T, D, H):
    x = x_ref[...]
    w1 = w1_ref[...]
    b1 = b1_ref[...]
    w2 = w2_ref[...]

    parts = []
    for t in range(T):
        xt = x[:, t * D:(t + 1) * D]
        h = jnp.dot(xt.astype(jnp.bfloat16), w1,
                    preferred_element_type=jnp.float32)
        h = jnp.maximum(h + b1, 0.0)
        hp = h.astype(jnp.bfloat16).astype(jnp.float32)
        parts.append(jnp.sum(hp * w2, axis=-1, keepdims=True))
    s = jnp.concatenate(parts, axis=-1)

    m = jnp.max(s, axis=-1, keepdims=True)
    e = jnp.exp(s - m)
    w = e / jnp.sum(e, axis=-1, keepdims=True)
    attw_ref[...] = w

    ctx = w[:, 0:1] * x[:, 0:D]
    for t in range(1, T):
        ctx = ctx + w[:, t:t + 1] * x[:, t * D:(t + 1) * D]
    ctx_ref[...] = ctx


def kernel(x, w1, b1, w2, b2, block_b=None):
    B, T, D = x.shape
    H = w1.shape[1]
    del b2

    x2 = x.reshape(B, T * D)
    w1b = w1.astype(jnp.bfloat16)
    b1r = b1.reshape(1, H).astype(jnp.float32)
    w2r = w2.reshape(1, H).astype(jnp.bfloat16).astype(jnp.float32)

    if block_b is None:
        block_b = 512
    if B >= 16:
        half = -(-B // 2)
        half = -(-half // 8) * 8
        block_b = min(block_b, half)
    block_b = max(8, block_b - block_b % 8)
    if block_b >= B:
        block_b = B
    n_blocks = pl.cdiv(B, block_b)

    ctx, attw = pl.pallas_call(
        functools.partial(_attn_kernel, T=T, D=D, H=H),
        out_shape=(
            jax.ShapeDtypeStruct((B, D), jnp.float32),
            jax.ShapeDtypeStruct((B, T), jnp.float32),
        ),
        grid_spec=pltpu.PrefetchScalarGridSpec(
            num_scalar_prefetch=0,
            grid=(n_blocks,),
            in_specs=[
                pl.BlockSpec((block_b, T * D), lambda b: (b, 0)),
                pl.BlockSpec((D, H), lambda b: (0, 0)),
                pl.BlockSpec((1, H), lambda b: (0, 0)),
                pl.BlockSpec((1, H), lambda b: (0, 0)),
            ],
            out_specs=[
                pl.BlockSpec((block_b, D), lambda b: (b, 0)),
                pl.BlockSpec((block_b, T), lambda b: (b, 0)),
            ],
        ),
        compiler_params=pltpu.CompilerParams(
            dimension_semantics=("parallel",),
            vmem_limit_bytes=64 * 1024 * 1024,
        ),
    )(x2, w1b, b1r, w2r)
    return ctx, attw


# speedup vs baseline: 1.7154x; 1.5987x over previous
import functools

import jax
import jax.numpy as jnp
from jax.experimental import pallas as pl
from jax.experimental.pallas import tpu as pltpu


def _attn_kernel(x_ref, w1_ref, b1_ref, w2_ref, ctx_ref, attw_ref, *, T, D, H):
    x3 = x_ref[...]
    TB = x3.shape[0]

    xa = x3.reshape(TB * T, D)
    h = jnp.dot(xa.astype(jnp.bfloat16), w1_ref[...],
                preferred_element_type=jnp.float32)
    h = jnp.maximum(h + b1_ref[...], 0.0)

    hp = h.astype(jnp.bfloat16).astype(jnp.float32)
    s = jnp.sum(hp * w2_ref[...], axis=-1, keepdims=True)
    s3 = s.reshape(TB, T, 1)

    m = jnp.max(s3, axis=1, keepdims=True)
    e = jnp.exp(s3 - m)
    w3 = e / jnp.sum(e, axis=1, keepdims=True)

    ctx_ref[...] = jnp.sum(w3 * x3, axis=1)

    attw_ref[...] = jnp.swapaxes(w3, 1, 2).reshape(TB, T)


def kernel(x, w1, b1, w2, b2, block_b=None):
    B, T, D = x.shape
    H = w1.shape[1]
    del b2

    w1b = w1.astype(jnp.bfloat16)
    b1r = b1.reshape(1, H).astype(jnp.float32)
    w2r = w2.reshape(1, H).astype(jnp.bfloat16).astype(jnp.float32)

    if block_b is None:
        block_b = 512
    if B >= 16:
        half = -(-B // 2)
        half = -(-half // 8) * 8
        block_b = min(block_b, half)
    block_b = max(8, block_b - block_b % 8)
    if block_b >= B:
        block_b = B
    n_blocks = pl.cdiv(B, block_b)

    ctx, attw = pl.pallas_call(
        functools.partial(_attn_kernel, T=T, D=D, H=H),
        out_shape=(
            jax.ShapeDtypeStruct((B, D), jnp.float32),
            jax.ShapeDtypeStruct((B, T), jnp.float32),
        ),
        grid_spec=pltpu.PrefetchScalarGridSpec(
            num_scalar_prefetch=0,
            grid=(n_blocks,),
            in_specs=[
                pl.BlockSpec((block_b, T, D), lambda b: (b, 0, 0)),
                pl.BlockSpec((D, H), lambda b: (0, 0)),
                pl.BlockSpec((1, H), lambda b: (0, 0)),
                pl.BlockSpec((1, H), lambda b: (0, 0)),
            ],
            out_specs=[
                pl.BlockSpec((block_b, D), lambda b: (b, 0)),
                pl.BlockSpec((block_b, T), lambda b: (b, 0)),
            ],
        ),
        compiler_params=pltpu.CompilerParams(
            dimension_semantics=("parallel",),
            vmem_limit_bytes=64 * 1024 * 1024,
        ),
    )(x, w1b, b1r, w2r)
    return ctx, attw


# speedup vs baseline: 1.9954x; 1.1632x over previous
import functools

import jax
import jax.numpy as jnp
from jax.experimental import pallas as pl
from jax.experimental.pallas import tpu as pltpu


def _attn_kernel(x_ref, w1_ref, b1_ref, w2_ref, ctx_ref, attw_ref, *, T, D, H):
    x3 = x_ref[...]
    TB = x3.shape[0]

    xa = x3.reshape(TB * T, D)
    h = jnp.dot(xa.astype(jnp.bfloat16), w1_ref[...],
                preferred_element_type=jnp.float32)
    h = jnp.maximum(h + b1_ref[...], 0.0)

    s = jnp.sum(h * w2_ref[...], axis=-1, keepdims=True)
    e3 = jnp.exp(s.reshape(TB, T, 1))

    ctx_un = jnp.sum(e3 * x3, axis=1)

    e_lane = jnp.swapaxes(e3, 1, 2).reshape(TB, T)
    denom = jnp.sum(e_lane, axis=-1, keepdims=True)
    r = 1.0 / denom
    attw_ref[...] = e_lane * r
    ctx_ref[...] = ctx_un * r


def kernel(x, w1, b1, w2, b2, block_b=None):
    B, T, D = x.shape
    H = w1.shape[1]
    del b2

    w1b = w1.astype(jnp.bfloat16)
    b1r = b1.reshape(1, H).astype(jnp.float32)
    w2r = w2.reshape(1, H).astype(jnp.bfloat16).astype(jnp.float32)

    if block_b is None:
        block_b = 512
    if B >= 16:
        half = -(-B // 2)
        half = -(-half // 8) * 8
        block_b = min(block_b, half)
    block_b = max(8, block_b - block_b % 8)
    if block_b >= B:
        block_b = B
    n_blocks = pl.cdiv(B, block_b)

    ctx, attw = pl.pallas_call(
        functools.partial(_attn_kernel, T=T, D=D, H=H),
        out_shape=(
            jax.ShapeDtypeStruct((B, D), jnp.float32),
            jax.ShapeDtypeStruct((B, T), jnp.float32),
        ),
        grid_spec=pltpu.PrefetchScalarGridSpec(
            num_scalar_prefetch=0,
            grid=(n_blocks,),
            in_specs=[
                pl.BlockSpec((block_b, T, D), lambda b: (b, 0, 0)),
                pl.BlockSpec((D, H), lambda b: (0, 0)),
                pl.BlockSpec((1, H), lambda b: (0, 0)),
                pl.BlockSpec((1, H), lambda b: (0, 0)),
            ],
            out_specs=[
                pl.BlockSpec((block_b, D), lambda b: (b, 0)),
                pl.BlockSpec((block_b, T), lambda b: (b, 0)),
            ],
        ),
        compiler_params=pltpu.CompilerParams(
            dimension_semantics=("parallel",),
            vmem_limit_bytes=64 * 1024 * 1024,
        ),
    )(x, w1b, b1r, w2r)
    return ctx, attw


# speedup vs baseline: 2.1119x; 1.0584x over previous
import functools

import jax
import jax.numpy as jnp
from jax.experimental import pallas as pl
from jax.experimental.pallas import tpu as pltpu


def _attn_kernel(x_ref, w1_ref, b1_ref, w2_ref, ctx_ref, attw_ref, *, T, D, H):
    x3 = x_ref[...]
    TB = x3.shape[0]

    xa = x3.reshape(TB * T, D)
    h = jnp.dot(xa.astype(jnp.bfloat16), w1_ref[...],
                preferred_element_type=jnp.float32)
    h = jnp.maximum(h + b1_ref[...], 0.0)

    s = jnp.sum(h * w2_ref[...], axis=-1, keepdims=True)
    e3 = jnp.exp(s.reshape(TB, T, 1))

    ctx_un = jnp.sum(e3 * x3, axis=1)

    e_lane = jnp.swapaxes(e3, 1, 2).reshape(TB, T)
    denom = jnp.sum(e_lane, axis=-1, keepdims=True)
    r = 1.0 / denom
    attw_ref[...] = e_lane * r
    ctx_ref[...] = ctx_un * r


def kernel(x, w1, b1, w2, b2, block_b=None):
    B, T, D = x.shape
    H = w1.shape[1]
    del b2

    w1b = w1.astype(jnp.bfloat16)
    b1r = b1.reshape(1, H).astype(jnp.float32)
    w2r = w2.reshape(1, H).astype(jnp.bfloat16).astype(jnp.float32)

    if block_b is None:
        block_b = 1024
    if B >= 16:
        half = -(-B // 2)
        half = -(-half // 8) * 8
        block_b = min(block_b, half)
    block_b = max(8, block_b - block_b % 8)
    if block_b >= B:
        block_b = B
    n_blocks = pl.cdiv(B, block_b)

    ctx, attw = pl.pallas_call(
        functools.partial(_attn_kernel, T=T, D=D, H=H),
        out_shape=(
            jax.ShapeDtypeStruct((B, D), jnp.float32),
            jax.ShapeDtypeStruct((B, T), jnp.float32),
        ),
        grid_spec=pltpu.PrefetchScalarGridSpec(
            num_scalar_prefetch=0,
            grid=(n_blocks,),
            in_specs=[
                pl.BlockSpec((block_b, T, D), lambda b: (b, 0, 0)),
                pl.BlockSpec((D, H), lambda b: (0, 0)),
                pl.BlockSpec((1, H), lambda b: (0, 0)),
                pl.BlockSpec((1, H), lambda b: (0, 0)),
            ],
            out_specs=[
                pl.BlockSpec((block_b, D), lambda b: (b, 0)),
                pl.BlockSpec((block_b, T), lambda b: (b, 0)),
            ],
        ),
        compiler_params=pltpu.CompilerParams(
            dimension_semantics=("parallel",),
            vmem_limit_bytes=64 * 1024 * 1024,
        ),
    )(x, w1b, b1r, w2r)
    return ctx, attw
